# Initial kernel scaffold; baseline (speedup 1.0000x reference)
import jax
import jax.numpy as jnp
from jax.experimental import pallas as pl


def kernel(x, Wg, bg, W1, b1, W2, b2):
    raise NotImplementedError("write your pallas kernel here")


# speedup vs baseline: 1.6897x; 1.6897x over previous
import jax
import jax.numpy as jnp
from jax.experimental import pallas as pl
from jax.experimental.pallas import tpu as pltpu
from jax.experimental.pallas import tpu_sc as plsc

D_MODEL = 1024
D_FF = 4096
NUM_EXPERTS = 8
NTOK = 2048
TM = 128
NTILES = NTOK // TM + NUM_EXPERTS - 1
NROWS = NTILES * TM


def _plan_body(x_ref, wg_ref, bg_ref, pos_ref, p_ref, te_ref, tv_ref, aux_ref):
    f32 = jnp.float32
    x = x_ref[...]
    wg = wg_ref[...]

    xh = x.astype(jnp.bfloat16)
    xl = (x - xh.astype(f32)).astype(jnp.bfloat16)
    wh = wg.astype(jnp.bfloat16)
    wl = (wg - wh.astype(f32)).astype(jnp.bfloat16)

    def dg(a, b):
        return jax.lax.dot_general(a, b, (((0,), (1,)), ((), ())),
                                   preferred_element_type=f32)

    lt = dg(wh, xh) + (dg(wh, xl) + dg(wl, xh))
    lt_route = dg(wh, xh)
    lt = lt + bg_ref[...]
    lt_route = lt_route + bg_ref[...]

    lmax = jnp.max(lt, axis=0, keepdims=True)
    denom = jnp.sum(jnp.exp(lt - lmax), axis=0, keepdims=True)
    p_ref[...] = 1.0 / denom

    si = jax.lax.broadcasted_iota(jnp.int32, (NUM_EXPERTS, NTOK), 0)
    lmax_r = jnp.max(lt_route, axis=0, keepdims=True)
    eidx = jnp.min(jnp.where(lt_route == lmax_r, si, NUM_EXPERTS), axis=0,
                   keepdims=True)
    oh = (si == eidx).astype(jnp.int32)

    c = oh
    s = 1
    while s < NTOK:
        c = c + jnp.concatenate(
            [jnp.zeros((NUM_EXPERTS, s), jnp.int32), c[:, :NTOK - s]], axis=1)
        s *= 2
    rank1 = jnp.sum(oh * c, axis=0, keepdims=True)

    g = jnp.sum(oh, axis=1, keepdims=True)
    pc = ((g + (TM - 1)) // TM) * TM
    po = pc
    t = 1
    while t < NUM_EXPERTS:
        po = po + jnp.concatenate(
            [jnp.zeros((t, 1), jnp.int32), po[:NUM_EXPERTS - t]], axis=0)
        t *= 2
    po = po - pc

    pos_ref[...] = jnp.sum(oh * po, axis=0, keepdims=True) + rank1 - 1

    kv = jax.lax.broadcasted_iota(jnp.int32, (NUM_EXPERTS, NTILES), 1) * TM
    cond = (kv >= po) & (kv < po + pc)
    ei = jax.lax.broadcasted_iota(jnp.int32, (NUM_EXPERTS, NTILES), 0)
    te = jnp.sum(jnp.where(cond, ei, 0), axis=0, keepdims=True)
    tv = jnp.sum(cond.astype(jnp.int32), axis=0, keepdims=True)
    elast = jnp.max(jnp.where(g > 0, ei[:, :1], -1), axis=0, keepdims=True)
    te_ref[...] = jnp.where(tv > 0, te, elast)
    tv_ref[...] = tv

    gf = g.astype(f32) * (1.0 / NTOK)
    aux_ref[...] = (jnp.sum(gf * gf) * NUM_EXPERTS).reshape(1, 1)


def _plan(xf, wg, bg2):
    return pl.pallas_call(
        _plan_body,
        out_shape=[
            jax.ShapeDtypeStruct((1, NTOK), jnp.int32),
            jax.ShapeDtypeStruct((1, NTOK), jnp.float32),
            jax.ShapeDtypeStruct((1, NTILES), jnp.int32),
            jax.ShapeDtypeStruct((1, NTILES), jnp.int32),
            jax.ShapeDtypeStruct((1, 1), jnp.float32),
        ],
    )(xf, wg, bg2)


_VMESH = None


def _vmesh():
    global _VMESH
    if _VMESH is None:
        _VMESH = plsc.VectorSubcoreMesh(core_axis_name="c", subcore_axis_name="s")
    return _VMESH


_NWORK = 32
_BPW = NTOK // _NWORK


def _dispatch(xbf, p16, pos):
    @pl.kernel(out_type=[jax.ShapeDtypeStruct((NROWS, D_MODEL), jnp.float32),
                         jax.ShapeDtypeStruct((NROWS, 128), jnp.float32)],
               mesh=_vmesh(),
               scratch_types=[pltpu.VMEM((_BPW,), jnp.int32),
                              pltpu.VMEM((_BPW, D_MODEL), jnp.float32),
                              pltpu.VMEM((_BPW, 128), jnp.float32),
                              pltpu.SemaphoreType.DMA,
                              pltpu.SemaphoreType.DMA])
    def k(x_hbm, p_hbm, i_hbm, xs_hbm, ps_hbm, idx_v, rows_v, pv, sem, sem2):
        wid = jax.lax.axis_index("s") * 2 + jax.lax.axis_index("c")
        base = wid * _BPW
        pltpu.sync_copy(i_hbm.at[pl.ds(base, _BPW)], idx_v)
        pltpu.sync_copy(x_hbm.at[pl.ds(base, _BPW)], rows_v)
        pltpu.sync_copy(p_hbm.at[pl.ds(base, _BPW)], pv)
        a = pltpu.async_copy(rows_v, xs_hbm.at[idx_v], sem)
        b = pltpu.async_copy(pv, ps_hbm.at[idx_v], sem2)
        a.wait()
        b.wait()

    return k(xbf, p16, pos)


def _combine(ys, pos):
    @pl.kernel(out_type=jax.ShapeDtypeStruct((NTOK, D_MODEL), jnp.float32),
               mesh=_vmesh(),
               scratch_types=[pltpu.VMEM((_BPW,), jnp.int32),
                              pltpu.VMEM((_BPW, D_MODEL), jnp.float32),
                              pltpu.SemaphoreType.DMA])
    def k(ys_hbm, i_hbm, o_hbm, idx_v, rows_v, sem):
        wid = jax.lax.axis_index("s") * 2 + jax.lax.axis_index("c")
        base = wid * _BPW
        pltpu.sync_copy(i_hbm.at[pl.ds(base, _BPW)], idx_v)
        pltpu.async_copy(ys_hbm.at[idx_v], rows_v, sem).wait()
        pltpu.sync_copy(rows_v, o_hbm.at[pl.ds(base, _BPW)])

    return k(ys, pos)


def _ffn_body(te_ref, tv_ref, xs_ref, w1_ref, b1_ref, w2_ref, b2_ref, ps_ref,
              ys_ref):
    k = pl.program_id(0)

    @pl.when(tv_ref[k] == 1)
    def _():
        h = jnp.dot(xs_ref[...].astype(jnp.bfloat16), w1_ref[0],
                    preferred_element_type=jnp.float32)
        h = jnp.maximum(h + b1_ref[0], 0.0)
        y = jnp.dot(h.astype(jnp.bfloat16), w2_ref[0],
                    preferred_element_type=jnp.float32)
        y = y + b2_ref[0]
        ys_ref[...] = y * ps_ref[:, 0:1]


def _ffn(te, tv, xs, w1, b1, w2, b2, ps):
    gs = pltpu.PrefetchScalarGridSpec(
        num_scalar_prefetch=2,
        grid=(NTILES,),
        in_specs=[
            pl.BlockSpec((TM, D_MODEL), lambda k, te, tv: (k, 0)),
            pl.BlockSpec((1, D_MODEL, D_FF), lambda k, te, tv: (te[k], 0, 0)),
            pl.BlockSpec((1, 1, D_FF), lambda k, te, tv: (te[k], 0, 0)),
            pl.BlockSpec((1, D_FF, D_MODEL), lambda k, te, tv: (te[k], 0, 0)),
            pl.BlockSpec((1, 1, D_MODEL), lambda k, te, tv: (te[k], 0, 0)),
            pl.BlockSpec((TM, 128), lambda k, te, tv: (k, 0)),
        ],
        out_specs=pl.BlockSpec((TM, D_MODEL), lambda k, te, tv: (k, 0)),
    )
    return pl.pallas_call(
        _ffn_body,
        grid_spec=gs,
        out_shape=jax.ShapeDtypeStruct((NROWS, D_MODEL), jnp.float32),
    )(te, tv, xs, w1, b1, w2, b2, ps)


def kernel(x, Wg, bg, W1, b1, W2, b2):
    xf = x.reshape(NTOK, D_MODEL)
    pos, p, te, tv, aux = _plan(xf, Wg, bg.reshape(NUM_EXPERTS, 1))
    te1 = te.reshape(NTILES)
    tv1 = tv.reshape(NTILES)
    p16 = jnp.broadcast_to(p.reshape(NTOK, 1), (NTOK, 128))
    xs, ps = _dispatch(xf, p16, pos.reshape(NTOK))
    ys = _ffn(te1, tv1, xs, W1.astype(jnp.bfloat16),
              b1.reshape(NUM_EXPERTS, 1, D_FF),
              W2.astype(jnp.bfloat16),
              b2.reshape(NUM_EXPERTS, 1, D_MODEL), ps)
    out = _combine(ys, pos.reshape(NTOK))
    return out.reshape(x.shape), aux.reshape(())


# speedup vs baseline: 1.7076x; 1.0106x over previous
import jax
import jax.numpy as jnp
from jax.experimental import pallas as pl
from jax.experimental.pallas import tpu as pltpu
from jax.experimental.pallas import tpu_sc as plsc

D_MODEL = 1024
D_FF = 4096
NUM_EXPERTS = 8
NTOK = 2048
TM = 128
NTILES = NTOK // TM + NUM_EXPERTS - 1
NROWS = NTILES * TM


def _plan_body(x_ref, wg_ref, bg_ref, pos_ref, p_ref, te_ref, tv_ref, aux_ref):
    f32 = jnp.float32
    x = x_ref[...]
    wg = wg_ref[...]

    xh = x.astype(jnp.bfloat16)
    xl = (x - xh.astype(f32)).astype(jnp.bfloat16)
    wh = wg.astype(jnp.bfloat16)
    wl = (wg - wh.astype(f32)).astype(jnp.bfloat16)

    def dg(a, b):
        return jax.lax.dot_general(a, b, (((0,), (1,)), ((), ())),
                                   preferred_element_type=f32)

    lt = dg(wh, xh) + (dg(wh, xl) + dg(wl, xh))
    lt_route = dg(wh, xh)
    lt = lt + bg_ref[...]
    lt_route = lt_route + bg_ref[...]

    lmax = jnp.max(lt, axis=0, keepdims=True)
    denom = jnp.sum(jnp.exp(lt - lmax), axis=0, keepdims=True)
    p_ref[...] = 1.0 / denom

    si = jax.lax.broadcasted_iota(jnp.int32, (NUM_EXPERTS, NTOK), 0)
    lmax_r = jnp.max(lt_route, axis=0, keepdims=True)
    eidx = jnp.min(jnp.where(lt_route == lmax_r, si, NUM_EXPERTS), axis=0,
                   keepdims=True)
    oh = (si == eidx).astype(jnp.int32)

    c = oh
    s = 1
    while s < NTOK:
        c = c + jnp.concatenate(
            [jnp.zeros((NUM_EXPERTS, s), jnp.int32), c[:, :NTOK - s]], axis=1)
        s *= 2
    rank1 = jnp.sum(oh * c, axis=0, keepdims=True)

    g = jnp.sum(oh, axis=1, keepdims=True)
    pc = ((g + (TM - 1)) // TM) * TM
    po = pc
    t = 1
    while t < NUM_EXPERTS:
        po = po + jnp.concatenate(
            [jnp.zeros((t, 1), jnp.int32), po[:NUM_EXPERTS - t]], axis=0)
        t *= 2
    po = po - pc

    pos_ref[...] = jnp.sum(oh * po, axis=0, keepdims=True) + rank1 - 1

    kv = jax.lax.broadcasted_iota(jnp.int32, (NUM_EXPERTS, NTILES), 1) * TM
    cond = (kv >= po) & (kv < po + pc)
    ei = jax.lax.broadcasted_iota(jnp.int32, (NUM_EXPERTS, NTILES), 0)
    te = jnp.sum(jnp.where(cond, ei, 0), axis=0, keepdims=True)
    tv = jnp.sum(cond.astype(jnp.int32), axis=0, keepdims=True)
    elast = jnp.max(jnp.where(g > 0, ei[:, :1], -1), axis=0, keepdims=True)
    te_ref[...] = jnp.where(tv > 0, te, elast)
    tv_ref[...] = tv

    gf = g.astype(f32) * (1.0 / NTOK)
    aux_ref[...] = (jnp.sum(gf * gf) * NUM_EXPERTS).reshape(1, 1)


def _plan(xf, wg, bg2):
    return pl.pallas_call(
        _plan_body,
        out_shape=[
            jax.ShapeDtypeStruct((1, NTOK), jnp.int32),
            jax.ShapeDtypeStruct((1, NTOK), jnp.float32),
            jax.ShapeDtypeStruct((1, NTILES), jnp.int32),
            jax.ShapeDtypeStruct((1, NTILES), jnp.int32),
            jax.ShapeDtypeStruct((1, 1), jnp.float32),
        ],
    )(xf, wg, bg2)


_VMESH = None


def _vmesh():
    global _VMESH
    if _VMESH is None:
        _VMESH = plsc.VectorSubcoreMesh(core_axis_name="c", subcore_axis_name="s")
    return _VMESH


_NWORK = 32
_BPW = NTOK // _NWORK


def _dispatch(xbf, p16, pos):
    @pl.kernel(out_type=[jax.ShapeDtypeStruct((NROWS, D_MODEL), jnp.float32),
                         jax.ShapeDtypeStruct((NROWS, 128), jnp.float32)],
               mesh=_vmesh(),
               scratch_types=[pltpu.VMEM((_BPW,), jnp.int32),
                              pltpu.VMEM((_BPW, D_MODEL), jnp.float32),
                              pltpu.VMEM((_BPW, 128), jnp.float32),
                              pltpu.SemaphoreType.DMA,
                              pltpu.SemaphoreType.DMA])
    def k(x_hbm, p_hbm, i_hbm, xs_hbm, ps_hbm, idx_v, rows_v, pv, sem, sem2):
        wid = jax.lax.axis_index("s") * 2 + jax.lax.axis_index("c")
        base = wid * _BPW
        pltpu.sync_copy(i_hbm.at[pl.ds(base, _BPW)], idx_v)
        pltpu.sync_copy(x_hbm.at[pl.ds(base, _BPW)], rows_v)
        pltpu.sync_copy(p_hbm.at[pl.ds(base, _BPW)], pv)
        a = pltpu.async_copy(rows_v, xs_hbm.at[idx_v], sem)
        b = pltpu.async_copy(pv, ps_hbm.at[idx_v], sem2)
        a.wait()
        b.wait()

    return k(xbf, p16, pos)


def _combine(ys, pos):
    @pl.kernel(out_type=jax.ShapeDtypeStruct((NTOK, D_MODEL), jnp.float32),
               mesh=_vmesh(),
               scratch_types=[pltpu.VMEM((_BPW,), jnp.int32),
                              pltpu.VMEM((_BPW, D_MODEL), jnp.float32),
                              pltpu.SemaphoreType.DMA])
    def k(ys_hbm, i_hbm, o_hbm, idx_v, rows_v, sem):
        wid = jax.lax.axis_index("s") * 2 + jax.lax.axis_index("c")
        base = wid * _BPW
        pltpu.sync_copy(i_hbm.at[pl.ds(base, _BPW)], idx_v)
        pltpu.async_copy(ys_hbm.at[idx_v], rows_v, sem).wait()
        pltpu.sync_copy(rows_v, o_hbm.at[pl.ds(base, _BPW)])

    return k(ys, pos)


FB = 1024


def _new_expert(te_ref, k):
    km1 = jnp.maximum(k - 1, 0)
    return (k == 0) | (te_ref[k] != te_ref[km1])


def _ffn1_body(te_ref, tv_ref, xs_ref, w1_ref, b1_ref, h_ref, w1bf):
    j = pl.program_id(0)
    k = pl.program_id(1)

    @pl.when(_new_expert(te_ref, k))
    def _():
        w1bf[...] = w1_ref[0].astype(jnp.bfloat16)

    @pl.when(tv_ref[k] == 1)
    def _():
        h = jnp.dot(xs_ref[...].astype(jnp.bfloat16), w1bf[...],
                    preferred_element_type=jnp.float32)
        h_ref[...] = jnp.maximum(h + b1_ref[0], 0.0).astype(jnp.bfloat16)


def _ffn1(te, tv, xs, w1, b1r):
    gs = pltpu.PrefetchScalarGridSpec(
        num_scalar_prefetch=2,
        grid=(D_FF // FB, NTILES),
        in_specs=[
            pl.BlockSpec((TM, D_MODEL), lambda j, k, te, tv: (k, 0)),
            pl.BlockSpec((1, D_MODEL, FB), lambda j, k, te, tv: (te[k], 0, j)),
            pl.BlockSpec((1, 1, FB), lambda j, k, te, tv: (te[k], 0, j)),
        ],
        out_specs=pl.BlockSpec((TM, FB), lambda j, k, te, tv: (k, j)),
        scratch_shapes=[pltpu.VMEM((D_MODEL, FB), jnp.bfloat16)],
    )
    return pl.pallas_call(
        _ffn1_body,
        grid_spec=gs,
        out_shape=jax.ShapeDtypeStruct((NROWS, D_FF), jnp.bfloat16),
    )(te, tv, xs, w1, b1r)


def _ffn2_body(te_ref, tv_ref, h_ref, w2_ref, b2_ref, ps_ref, ys_ref, w2bf):
    k = pl.program_id(0)

    @pl.when(_new_expert(te_ref, k))
    def _():
        w2bf[...] = w2_ref[0].astype(jnp.bfloat16)

    @pl.when(tv_ref[k] == 1)
    def _():
        y = jnp.dot(h_ref[...], w2bf[...], preferred_element_type=jnp.float32)
        y = y + b2_ref[0]
        ys_ref[...] = y * ps_ref[:, 0:1]


def _ffn2(te, tv, h, w2, b2r, ps):
    gs = pltpu.PrefetchScalarGridSpec(
        num_scalar_prefetch=2,
        grid=(NTILES,),
        in_specs=[
            pl.BlockSpec((TM, D_FF), lambda k, te, tv: (k, 0)),
            pl.BlockSpec((1, D_FF, D_MODEL), lambda k, te, tv: (te[k], 0, 0)),
            pl.BlockSpec((1, 1, D_MODEL), lambda k, te, tv: (te[k], 0, 0)),
            pl.BlockSpec((TM, 128), lambda k, te, tv: (k, 0)),
        ],
        out_specs=pl.BlockSpec((TM, D_MODEL), lambda k, te, tv: (k, 0)),
        scratch_shapes=[pltpu.VMEM((D_FF, D_MODEL), jnp.bfloat16)],
    )
    return pl.pallas_call(
        _ffn2_body,
        grid_spec=gs,
        out_shape=jax.ShapeDtypeStruct((NROWS, D_MODEL), jnp.float32),
    )(te, tv, h, w2, b2r, ps)


def kernel(x, Wg, bg, W1, b1, W2, b2):
    xf = x.reshape(NTOK, D_MODEL)
    pos, p, te, tv, aux = _plan(xf, Wg, bg.reshape(NUM_EXPERTS, 1))
    te1 = te.reshape(NTILES)
    tv1 = tv.reshape(NTILES)
    p16 = jnp.broadcast_to(p.reshape(NTOK, 1), (NTOK, 128))
    xs, ps = _dispatch(xf, p16, pos.reshape(NTOK))
    h = _ffn1(te1, tv1, xs, W1, b1.reshape(NUM_EXPERTS, 1, D_FF))
    ys = _ffn2(te1, tv1, h, W2, b2.reshape(NUM_EXPERTS, 1, D_MODEL), ps)
    out = _combine(ys, pos.reshape(NTOK))
    return out.reshape(x.shape), aux.reshape(())


# speedup vs baseline: 1.9277x; 1.1289x over previous
import jax
import jax.numpy as jnp
from jax.experimental import pallas as pl
from jax.experimental.pallas import tpu as pltpu
from jax.experimental.pallas import tpu_sc as plsc

D_MODEL = 1024
D_FF = 4096
NUM_EXPERTS = 8
NTOK = 2048
TM = 128
NTILES = NTOK // TM + NUM_EXPERTS - 1
NROWS = NTILES * TM


def _plan_body(x_ref, wg_ref, bg_ref, pos_ref, p_ref, te_ref, tv_ref, aux_ref):
    f32 = jnp.float32
    x = x_ref[...]
    wg = wg_ref[...]

    xh = x.astype(jnp.bfloat16)
    xl = (x - xh.astype(f32)).astype(jnp.bfloat16)
    wh = wg.astype(jnp.bfloat16)
    wl = (wg - wh.astype(f32)).astype(jnp.bfloat16)

    def dg(a, b):
        return jax.lax.dot_general(a, b, (((0,), (1,)), ((), ())),
                                   preferred_element_type=f32)

    lt = dg(wh, xh) + (dg(wh, xl) + dg(wl, xh))
    lt_route = dg(wh, xh)
    lt = lt + bg_ref[...]
    lt_route = lt_route + bg_ref[...]

    lmax = jnp.max(lt, axis=0, keepdims=True)
    denom = jnp.sum(jnp.exp(lt - lmax), axis=0, keepdims=True)
    p_ref[...] = 1.0 / denom

    si = jax.lax.broadcasted_iota(jnp.int32, (NUM_EXPERTS, NTOK), 0)
    lmax_r = jnp.max(lt_route, axis=0, keepdims=True)
    eidx = jnp.min(jnp.where(lt_route == lmax_r, si, NUM_EXPERTS), axis=0,
                   keepdims=True)
    oh = (si == eidx).astype(jnp.int32)

    c = oh
    s = 1
    while s < NTOK:
        c = c + jnp.concatenate(
            [jnp.zeros((NUM_EXPERTS, s), jnp.int32), c[:, :NTOK - s]], axis=1)
        s *= 2
    rank1 = jnp.sum(oh * c, axis=0, keepdims=True)

    g = jnp.sum(oh, axis=1, keepdims=True)
    pc = ((g + (TM - 1)) // TM) * TM
    po = pc
    t = 1
    while t < NUM_EXPERTS:
        po = po + jnp.concatenate(
            [jnp.zeros((t, 1), jnp.int32), po[:NUM_EXPERTS - t]], axis=0)
        t *= 2
    po = po - pc

    pos_ref[...] = jnp.sum(oh * po, axis=0, keepdims=True) + rank1 - 1

    kv = jax.lax.broadcasted_iota(jnp.int32, (NUM_EXPERTS, NTILES), 1) * TM
    cond = (kv >= po) & (kv < po + pc)
    ei = jax.lax.broadcasted_iota(jnp.int32, (NUM_EXPERTS, NTILES), 0)
    te = jnp.sum(jnp.where(cond, ei, 0), axis=0, keepdims=True)
    tv = jnp.sum(cond.astype(jnp.int32), axis=0, keepdims=True)
    elast = jnp.max(jnp.where(g > 0, ei[:, :1], -1), axis=0, keepdims=True)
    te_ref[...] = jnp.where(tv > 0, te, elast)
    tv_ref[...] = tv

    gf = g.astype(f32) * (1.0 / NTOK)
    aux_ref[...] = (jnp.sum(gf * gf) * NUM_EXPERTS).reshape(1, 1)


def _plan(xf, wg, bg2):
    return pl.pallas_call(
        _plan_body,
        out_shape=[
            jax.ShapeDtypeStruct((1, NTOK), jnp.int32),
            jax.ShapeDtypeStruct((1, NTOK), jnp.float32),
            jax.ShapeDtypeStruct((1, NTILES), jnp.int32),
            jax.ShapeDtypeStruct((1, NTILES), jnp.int32),
            jax.ShapeDtypeStruct((1, 1), jnp.float32),
        ],
    )(xf, wg, bg2)


_VMESH = None


def _vmesh():
    global _VMESH
    if _VMESH is None:
        _VMESH = plsc.VectorSubcoreMesh(core_axis_name="c", subcore_axis_name="s")
    return _VMESH


_NWORK = 32
_BPW = NTOK // _NWORK


def _dispatch(xbf, p16, pos):
    @pl.kernel(out_type=[jax.ShapeDtypeStruct((NROWS, D_MODEL), jnp.float32),
                         jax.ShapeDtypeStruct((NROWS, 128), jnp.float32)],
               mesh=_vmesh(),
               scratch_types=[pltpu.VMEM((_BPW,), jnp.int32),
                              pltpu.VMEM((_BPW, D_MODEL), jnp.float32),
                              pltpu.VMEM((_BPW, 128), jnp.float32),
                              pltpu.SemaphoreType.DMA,
                              pltpu.SemaphoreType.DMA])
    def k(x_hbm, p_hbm, i_hbm, xs_hbm, ps_hbm, idx_v, rows_v, pv, sem, sem2):
        wid = jax.lax.axis_index("s") * 2 + jax.lax.axis_index("c")
        base = wid * _BPW
        pltpu.sync_copy(i_hbm.at[pl.ds(base, _BPW)], idx_v)
        pltpu.sync_copy(x_hbm.at[pl.ds(base, _BPW)], rows_v)
        pltpu.sync_copy(p_hbm.at[pl.ds(base, _BPW)], pv)
        a = pltpu.async_copy(rows_v, xs_hbm.at[idx_v], sem)
        b = pltpu.async_copy(pv, ps_hbm.at[idx_v], sem2)
        a.wait()
        b.wait()

    return k(xbf, p16, pos)


def _combine(ys, pos):
    @pl.kernel(out_type=jax.ShapeDtypeStruct((NTOK, D_MODEL), jnp.float32),
               mesh=_vmesh(),
               scratch_types=[pltpu.VMEM((_BPW,), jnp.int32),
                              pltpu.VMEM((_BPW, D_MODEL), jnp.float32),
                              pltpu.SemaphoreType.DMA])
    def k(ys_hbm, i_hbm, o_hbm, idx_v, rows_v, sem):
        wid = jax.lax.axis_index("s") * 2 + jax.lax.axis_index("c")
        base = wid * _BPW
        pltpu.sync_copy(i_hbm.at[pl.ds(base, _BPW)], idx_v)
        pltpu.async_copy(ys_hbm.at[idx_v], rows_v, sem).wait()
        pltpu.sync_copy(rows_v, o_hbm.at[pl.ds(base, _BPW)])

    return k(ys, pos)


FB = 2048


def _new_expert(te_ref, k):
    km1 = jnp.maximum(k - 1, 0)
    return (k == 0) | (te_ref[k] != te_ref[km1])


def _ffn1_body(te_ref, tv_ref, xs_ref, w1_ref, b1_ref, h_ref, w1bf):
    j = pl.program_id(0)
    k = pl.program_id(1)

    @pl.when(_new_expert(te_ref, k))
    def _():
        w1bf[...] = w1_ref[0].astype(jnp.bfloat16)

    @pl.when(tv_ref[k] == 1)
    def _():
        h = jnp.dot(xs_ref[...].astype(jnp.bfloat16), w1bf[...],
                    preferred_element_type=jnp.float32)
        h_ref[...] = jnp.maximum(h + b1_ref[0], 0.0).astype(jnp.bfloat16)


def _ffn1(te, tv, xs, w1, b1r):
    gs = pltpu.PrefetchScalarGridSpec(
        num_scalar_prefetch=2,
        grid=(D_FF // FB, NTILES),
        in_specs=[
            pl.BlockSpec((TM, D_MODEL), lambda j, k, te, tv: (k, 0)),
            pl.BlockSpec((1, D_MODEL, FB), lambda j, k, te, tv: (te[k], 0, j)),
            pl.BlockSpec((1, 1, FB), lambda j, k, te, tv: (te[k], 0, j)),
        ],
        out_specs=pl.BlockSpec((TM, FB), lambda j, k, te, tv: (k, j)),
        scratch_shapes=[pltpu.VMEM((D_MODEL, FB), jnp.bfloat16)],
    )
    return pl.pallas_call(
        _ffn1_body,
        grid_spec=gs,
        out_shape=jax.ShapeDtypeStruct((NROWS, D_FF), jnp.bfloat16),
    )(te, tv, xs, w1, b1r)


def _ffn2_body(te_ref, tv_ref, h_ref, w2_ref, b2_ref, ps_ref, ys_ref, w2bf):
    k = pl.program_id(0)

    @pl.when(_new_expert(te_ref, k))
    def _():
        w2bf[...] = w2_ref[0].astype(jnp.bfloat16)

    @pl.when(tv_ref[k] == 1)
    def _():
        y = jnp.dot(h_ref[...], w2bf[...], preferred_element_type=jnp.float32)
        y = y + b2_ref[0]
        ys_ref[...] = y * ps_ref[:, 0:1]


def _ffn2(te, tv, h, w2, b2r, ps):
    gs = pltpu.PrefetchScalarGridSpec(
        num_scalar_prefetch=2,
        grid=(NTILES,),
        in_specs=[
            pl.BlockSpec((TM, D_FF), lambda k, te, tv: (k, 0)),
            pl.BlockSpec((1, D_FF, D_MODEL), lambda k, te, tv: (te[k], 0, 0)),
            pl.BlockSpec((1, 1, D_MODEL), lambda k, te, tv: (te[k], 0, 0)),
            pl.BlockSpec((TM, 128), lambda k, te, tv: (k, 0)),
        ],
        out_specs=pl.BlockSpec((TM, D_MODEL), lambda k, te, tv: (k, 0)),
        scratch_shapes=[pltpu.VMEM((D_FF, D_MODEL), jnp.bfloat16)],
    )
    return pl.pallas_call(
        _ffn2_body,
        grid_spec=gs,
        out_shape=jax.ShapeDtypeStruct((NROWS, D_MODEL), jnp.float32),
    )(te, tv, h, w2, b2r, ps)


def kernel(x, Wg, bg, W1, b1, W2, b2):
    xf = x.reshape(NTOK, D_MODEL)
    pos, p, te, tv, aux = _plan(xf, Wg, bg.reshape(NUM_EXPERTS, 1))
    te1 = te.reshape(NTILES)
    tv1 = tv.reshape(NTILES)
    p16 = jnp.broadcast_to(p.reshape(NTOK, 1), (NTOK, 128))
    xs, ps = _dispatch(xf, p16, pos.reshape(NTOK))
    h = _ffn1(te1, tv1, xs, W1, b1.reshape(NUM_EXPERTS, 1, D_FF))
    ys = _ffn2(te1, tv1, h, W2, b2.reshape(NUM_EXPERTS, 1, D_MODEL), ps)
    out = _combine(ys, pos.reshape(NTOK))
    return out.reshape(x.shape), aux.reshape(())


# speedup vs baseline: 2.0625x; 1.0699x over previous
import jax
import jax.numpy as jnp
from jax.experimental import pallas as pl
from jax.experimental.pallas import tpu as pltpu
from jax.experimental.pallas import tpu_sc as plsc

D_MODEL = 1024
D_FF = 4096
NUM_EXPERTS = 8
NTOK = 2048
TM = 128
NTILES = NTOK // TM + NUM_EXPERTS - 1
NROWS = NTILES * TM


def _plan_body(x_ref, wg_ref, bg_ref, pos_ref, p_ref, te_ref, tv_ref, aux_ref):
    f32 = jnp.float32
    x = x_ref[...]
    wg = wg_ref[...]

    xh = x.astype(jnp.bfloat16)
    xl = (x - xh.astype(f32)).astype(jnp.bfloat16)
    wh = wg.astype(jnp.bfloat16)
    wl = (wg - wh.astype(f32)).astype(jnp.bfloat16)

    def dg(a, b):
        return jax.lax.dot_general(a, b, (((0,), (1,)), ((), ())),
                                   preferred_element_type=f32)

    lt = dg(wh, xh) + (dg(wh, xl) + dg(wl, xh))
    lt_route = dg(wh, xh)
    lt = lt + bg_ref[...]
    lt_route = lt_route + bg_ref[...]

    lmax = jnp.max(lt, axis=0, keepdims=True)
    denom = jnp.sum(jnp.exp(lt - lmax), axis=0, keepdims=True)
    p_ref[...] = 1.0 / denom

    si = jax.lax.broadcasted_iota(jnp.int32, (NUM_EXPERTS, NTOK), 0)
    lmax_r = jnp.max(lt_route, axis=0, keepdims=True)
    eidx = jnp.min(jnp.where(lt_route == lmax_r, si, NUM_EXPERTS), axis=0,
                   keepdims=True)
    oh = (si == eidx).astype(jnp.int32)

    c = oh
    s = 1
    while s < NTOK:
        c = c + jnp.concatenate(
            [jnp.zeros((NUM_EXPERTS, s), jnp.int32), c[:, :NTOK - s]], axis=1)
        s *= 2
    rank1 = jnp.sum(oh * c, axis=0, keepdims=True)

    g = jnp.sum(oh, axis=1, keepdims=True)
    pc = ((g + (TM - 1)) // TM) * TM
    po = pc
    t = 1
    while t < NUM_EXPERTS:
        po = po + jnp.concatenate(
            [jnp.zeros((t, 1), jnp.int32), po[:NUM_EXPERTS - t]], axis=0)
        t *= 2
    po = po - pc

    pos_ref[...] = jnp.sum(oh * po, axis=0, keepdims=True) + rank1 - 1

    kv = jax.lax.broadcasted_iota(jnp.int32, (NUM_EXPERTS, NTILES), 1) * TM
    cond = (kv >= po) & (kv < po + pc)
    ei = jax.lax.broadcasted_iota(jnp.int32, (NUM_EXPERTS, NTILES), 0)
    te = jnp.sum(jnp.where(cond, ei, 0), axis=0, keepdims=True)
    tv = jnp.sum(cond.astype(jnp.int32), axis=0, keepdims=True)
    elast = jnp.max(jnp.where(g > 0, ei[:, :1], -1), axis=0, keepdims=True)
    te_ref[...] = jnp.where(tv > 0, te, elast)
    tv_ref[...] = tv

    gf = g.astype(f32) * (1.0 / NTOK)
    aux_ref[...] = (jnp.sum(gf * gf) * NUM_EXPERTS).reshape(1, 1)


def _plan(xf, wg, bg2):
    return pl.pallas_call(
        _plan_body,
        out_shape=[
            jax.ShapeDtypeStruct((1, NTOK), jnp.int32),
            jax.ShapeDtypeStruct((1, NTOK), jnp.float32),
            jax.ShapeDtypeStruct((1, NTILES), jnp.int32),
            jax.ShapeDtypeStruct((1, NTILES), jnp.int32),
            jax.ShapeDtypeStruct((1, 1), jnp.float32),
        ],
    )(xf, wg, bg2)


_VMESH = None


def _vmesh():
    global _VMESH
    if _VMESH is None:
        _VMESH = plsc.VectorSubcoreMesh(core_axis_name="c", subcore_axis_name="s")
    return _VMESH


_NWORK = 32
_BPW = NTOK // _NWORK


def _dispatch(xbf, p16, pos):
    @pl.kernel(out_type=[jax.ShapeDtypeStruct((NROWS, D_MODEL), jnp.float32),
                         jax.ShapeDtypeStruct((NROWS, 128), jnp.float32)],
               mesh=_vmesh(),
               scratch_types=[pltpu.VMEM((_BPW,), jnp.int32),
                              pltpu.VMEM((_BPW, D_MODEL), jnp.float32),
                              pltpu.VMEM((_BPW, 128), jnp.float32),
                              pltpu.SemaphoreType.DMA,
                              pltpu.SemaphoreType.DMA])
    def k(x_hbm, p_hbm, i_hbm, xs_hbm, ps_hbm, idx_v, rows_v, pv, sem, sem2):
        wid = jax.lax.axis_index("s") * 2 + jax.lax.axis_index("c")
        base = wid * _BPW
        pltpu.sync_copy(i_hbm.at[pl.ds(base, _BPW)], idx_v)
        pltpu.sync_copy(x_hbm.at[pl.ds(base, _BPW)], rows_v)
        pltpu.sync_copy(p_hbm.at[pl.ds(base, _BPW)], pv)
        a = pltpu.async_copy(rows_v, xs_hbm.at[idx_v], sem)
        b = pltpu.async_copy(pv, ps_hbm.at[idx_v], sem2)
        a.wait()
        b.wait()

    return k(xbf, p16, pos)


def _combine(ys, pos):
    @pl.kernel(out_type=jax.ShapeDtypeStruct((NTOK, D_MODEL), jnp.float32),
               mesh=_vmesh(),
               scratch_types=[pltpu.VMEM((_BPW,), jnp.int32),
                              pltpu.VMEM((_BPW, D_MODEL), jnp.float32),
                              pltpu.SemaphoreType.DMA])
    def k(ys_hbm, i_hbm, o_hbm, idx_v, rows_v, sem):
        wid = jax.lax.axis_index("s") * 2 + jax.lax.axis_index("c")
        base = wid * _BPW
        pltpu.sync_copy(i_hbm.at[pl.ds(base, _BPW)], idx_v)
        pltpu.async_copy(ys_hbm.at[idx_v], rows_v, sem).wait()
        pltpu.sync_copy(rows_v, o_hbm.at[pl.ds(base, _BPW)])

    return k(ys, pos)


FB = 4096


def _new_expert(te_ref, k):
    km1 = jnp.maximum(k - 1, 0)
    return (k == 0) | (te_ref[k] != te_ref[km1])


def _ffn1_body(te_ref, tv_ref, xs_ref, w1_ref, b1_ref, h_ref, w1bf):
    j = pl.program_id(0)
    k = pl.program_id(1)

    @pl.when(_new_expert(te_ref, k))
    def _():
        w1bf[...] = w1_ref[0].astype(jnp.bfloat16)

    @pl.when(tv_ref[k] == 1)
    def _():
        h = jnp.dot(xs_ref[...].astype(jnp.bfloat16), w1bf[...],
                    preferred_element_type=jnp.float32)
        h_ref[...] = jnp.maximum(h + b1_ref[0], 0.0).astype(jnp.bfloat16)


def _ffn1(te, tv, xs, w1, b1r):
    gs = pltpu.PrefetchScalarGridSpec(
        num_scalar_prefetch=2,
        grid=(D_FF // FB, NTILES),
        in_specs=[
            pl.BlockSpec((TM, D_MODEL), lambda j, k, te, tv: (k, 0)),
            pl.BlockSpec((1, D_MODEL, FB), lambda j, k, te, tv: (te[k], 0, j)),
            pl.BlockSpec((1, 1, FB), lambda j, k, te, tv: (te[k], 0, j)),
        ],
        out_specs=pl.BlockSpec((TM, FB), lambda j, k, te, tv: (k, j)),
        scratch_shapes=[pltpu.VMEM((D_MODEL, FB), jnp.bfloat16)],
    )
    return pl.pallas_call(
        _ffn1_body,
        grid_spec=gs,
        out_shape=jax.ShapeDtypeStruct((NROWS, D_FF), jnp.bfloat16),
    )(te, tv, xs, w1, b1r)


def _ffn2_body(te_ref, tv_ref, h_ref, w2_ref, b2_ref, ps_ref, ys_ref, w2bf):
    k = pl.program_id(0)

    @pl.when(_new_expert(te_ref, k))
    def _():
        w2bf[...] = w2_ref[0].astype(jnp.bfloat16)

    @pl.when(tv_ref[k] == 1)
    def _():
        y = jnp.dot(h_ref[...], w2bf[...], preferred_element_type=jnp.float32)
        y = y + b2_ref[0]
        ys_ref[...] = y * ps_ref[:, 0:1]


def _ffn2(te, tv, h, w2, b2r, ps):
    gs = pltpu.PrefetchScalarGridSpec(
        num_scalar_prefetch=2,
        grid=(NTILES,),
        in_specs=[
            pl.BlockSpec((TM, D_FF), lambda k, te, tv: (k, 0)),
            pl.BlockSpec((1, D_FF, D_MODEL), lambda k, te, tv: (te[k], 0, 0)),
            pl.BlockSpec((1, 1, D_MODEL), lambda k, te, tv: (te[k], 0, 0)),
            pl.BlockSpec((TM, 128), lambda k, te, tv: (k, 0)),
        ],
        out_specs=pl.BlockSpec((TM, D_MODEL), lambda k, te, tv: (k, 0)),
        scratch_shapes=[pltpu.VMEM((D_FF, D_MODEL), jnp.bfloat16)],
    )
    return pl.pallas_call(
        _ffn2_body,
        grid_spec=gs,
        out_shape=jax.ShapeDtypeStruct((NROWS, D_MODEL), jnp.float32),
    )(te, tv, h, w2, b2r, ps)


def kernel(x, Wg, bg, W1, b1, W2, b2):
    xf = x.reshape(NTOK, D_MODEL)
    pos, p, te, tv, aux = _plan(xf, Wg, bg.reshape(NUM_EXPERTS, 1))
    te1 = te.reshape(NTILES)
    tv1 = tv.reshape(NTILES)
    p16 = jnp.broadcast_to(p.reshape(NTOK, 1), (NTOK, 128))
    xs, ps = _dispatch(xf, p16, pos.reshape(NTOK))
    h = _ffn1(te1, tv1, xs, W1, b1.reshape(NUM_EXPERTS, 1, D_FF))
    ys = _ffn2(te1, tv1, h, W2, b2.reshape(NUM_EXPERTS, 1, D_MODEL), ps)
    out = _combine(ys, pos.reshape(NTOK))
    return out.reshape(x.shape), aux.reshape(())


# speedup vs baseline: 2.1501x; 1.0425x over previous
import jax
import jax.numpy as jnp
from jax.experimental import pallas as pl
from jax.experimental.pallas import tpu as pltpu
from jax.experimental.pallas import tpu_sc as plsc

D_MODEL = 1024
D_FF = 4096
NUM_EXPERTS = 8
NTOK = 2048
TM = 256
NTILES = NTOK // TM + NUM_EXPERTS - 1
NROWS = NTILES * TM


def _plan_body(x_ref, wg_ref, bg_ref, pos_ref, p_ref, te_ref, tv_ref, aux_ref):
    f32 = jnp.float32
    x = x_ref[...]
    wg = wg_ref[...]

    xh = x.astype(jnp.bfloat16)
    xl = (x - xh.astype(f32)).astype(jnp.bfloat16)
    wh = wg.astype(jnp.bfloat16)
    wl = (wg - wh.astype(f32)).astype(jnp.bfloat16)

    def dg(a, b):
        return jax.lax.dot_general(a, b, (((0,), (1,)), ((), ())),
                                   preferred_element_type=f32)

    lt = dg(wh, xh) + (dg(wh, xl) + dg(wl, xh))
    lt_route = dg(wh, xh)
    lt = lt + bg_ref[...]
    lt_route = lt_route + bg_ref[...]

    lmax = jnp.max(lt, axis=0, keepdims=True)
    denom = jnp.sum(jnp.exp(lt - lmax), axis=0, keepdims=True)
    p_ref[...] = 1.0 / denom

    si = jax.lax.broadcasted_iota(jnp.int32, (NUM_EXPERTS, NTOK), 0)
    lmax_r = jnp.max(lt_route, axis=0, keepdims=True)
    eidx = jnp.min(jnp.where(lt_route == lmax_r, si, NUM_EXPERTS), axis=0,
                   keepdims=True)
    oh = (si == eidx).astype(jnp.int32)

    c = oh
    s = 1
    while s < NTOK:
        c = c + jnp.concatenate(
            [jnp.zeros((NUM_EXPERTS, s), jnp.int32), c[:, :NTOK - s]], axis=1)
        s *= 2
    rank1 = jnp.sum(oh * c, axis=0, keepdims=True)

    g = jnp.sum(oh, axis=1, keepdims=True)
    pc = ((g + (TM - 1)) // TM) * TM
    po = pc
    t = 1
    while t < NUM_EXPERTS:
        po = po + jnp.concatenate(
            [jnp.zeros((t, 1), jnp.int32), po[:NUM_EXPERTS - t]], axis=0)
        t *= 2
    po = po - pc

    pos_ref[...] = jnp.sum(oh * po, axis=0, keepdims=True) + rank1 - 1

    kv = jax.lax.broadcasted_iota(jnp.int32, (NUM_EXPERTS, NTILES), 1) * TM
    cond = (kv >= po) & (kv < po + pc)
    ei = jax.lax.broadcasted_iota(jnp.int32, (NUM_EXPERTS, NTILES), 0)
    te = jnp.sum(jnp.where(cond, ei, 0), axis=0, keepdims=True)
    tv = jnp.sum(cond.astype(jnp.int32), axis=0, keepdims=True)
    elast = jnp.max(jnp.where(g > 0, ei[:, :1], -1), axis=0, keepdims=True)
    te_ref[...] = jnp.where(tv > 0, te, elast)
    tv_ref[...] = tv

    gf = g.astype(f32) * (1.0 / NTOK)
    aux_ref[...] = (jnp.sum(gf * gf) * NUM_EXPERTS).reshape(1, 1)


def _plan(xf, wg, bg2):
    return pl.pallas_call(
        _plan_body,
        out_shape=[
            jax.ShapeDtypeStruct((1, NTOK), jnp.int32),
            jax.ShapeDtypeStruct((1, NTOK), jnp.float32),
            jax.ShapeDtypeStruct((1, NTILES), jnp.int32),
            jax.ShapeDtypeStruct((1, NTILES), jnp.int32),
            jax.ShapeDtypeStruct((1, 1), jnp.float32),
        ],
    )(xf, wg, bg2)


_VMESH = None


def _vmesh():
    global _VMESH
    if _VMESH is None:
        _VMESH = plsc.VectorSubcoreMesh(core_axis_name="c", subcore_axis_name="s")
    return _VMESH


_NWORK = 32
_BPW = NTOK // _NWORK


def _dispatch(xbf, p16, pos):
    @pl.kernel(out_type=[jax.ShapeDtypeStruct((NROWS, D_MODEL), jnp.float32),
                         jax.ShapeDtypeStruct((NROWS, 128), jnp.float32)],
               mesh=_vmesh(),
               scratch_types=[pltpu.VMEM((_BPW,), jnp.int32),
                              pltpu.VMEM((_BPW, D_MODEL), jnp.float32),
                              pltpu.VMEM((_BPW, 128), jnp.float32),
                              pltpu.SemaphoreType.DMA,
                              pltpu.SemaphoreType.DMA])
    def k(x_hbm, p_hbm, i_hbm, xs_hbm, ps_hbm, idx_v, rows_v, pv, sem, sem2):
        wid = jax.lax.axis_index("s") * 2 + jax.lax.axis_index("c")
        base = wid * _BPW
        pltpu.sync_copy(i_hbm.at[pl.ds(base, _BPW)], idx_v)
        pltpu.sync_copy(x_hbm.at[pl.ds(base, _BPW)], rows_v)
        pltpu.sync_copy(p_hbm.at[pl.ds(base, _BPW)], pv)
        a = pltpu.async_copy(rows_v, xs_hbm.at[idx_v], sem)
        b = pltpu.async_copy(pv, ps_hbm.at[idx_v], sem2)
        a.wait()
        b.wait()

    return k(xbf, p16, pos)


def _combine(ys, pos):
    @pl.kernel(out_type=jax.ShapeDtypeStruct((NTOK, D_MODEL), jnp.float32),
               mesh=_vmesh(),
               scratch_types=[pltpu.VMEM((_BPW,), jnp.int32),
                              pltpu.VMEM((_BPW, D_MODEL), jnp.float32),
                              pltpu.SemaphoreType.DMA])
    def k(ys_hbm, i_hbm, o_hbm, idx_v, rows_v, sem):
        wid = jax.lax.axis_index("s") * 2 + jax.lax.axis_index("c")
        base = wid * _BPW
        pltpu.sync_copy(i_hbm.at[pl.ds(base, _BPW)], idx_v)
        pltpu.async_copy(ys_hbm.at[idx_v], rows_v, sem).wait()
        pltpu.sync_copy(rows_v, o_hbm.at[pl.ds(base, _BPW)])

    return k(ys, pos)


FB = 4096


def _new_expert(te_ref, k):
    km1 = jnp.maximum(k - 1, 0)
    return (k == 0) | (te_ref[k] != te_ref[km1])


def _ffn1_body(te_ref, tv_ref, xs_ref, w1_ref, b1_ref, h_ref, w1bf):
    j = pl.program_id(0)
    k = pl.program_id(1)

    @pl.when(_new_expert(te_ref, k))
    def _():
        w1bf[...] = w1_ref[0].astype(jnp.bfloat16)

    @pl.when(tv_ref[k] == 1)
    def _():
        h = jnp.dot(xs_ref[...].astype(jnp.bfloat16), w1bf[...],
                    preferred_element_type=jnp.float32)
        h_ref[...] = jnp.maximum(h + b1_ref[0], 0.0).astype(jnp.bfloat16)


def _ffn1(te, tv, xs, w1, b1r):
    gs = pltpu.PrefetchScalarGridSpec(
        num_scalar_prefetch=2,
        grid=(D_FF // FB, NTILES),
        in_specs=[
            pl.BlockSpec((TM, D_MODEL), lambda j, k, te, tv: (k, 0)),
            pl.BlockSpec((1, D_MODEL, FB), lambda j, k, te, tv: (te[k], 0, j)),
            pl.BlockSpec((1, 1, FB), lambda j, k, te, tv: (te[k], 0, j)),
        ],
        out_specs=pl.BlockSpec((TM, FB), lambda j, k, te, tv: (k, j)),
        scratch_shapes=[pltpu.VMEM((D_MODEL, FB), jnp.bfloat16)],
    )
    return pl.pallas_call(
        _ffn1_body,
        grid_spec=gs,
        out_shape=jax.ShapeDtypeStruct((NROWS, D_FF), jnp.bfloat16),
    )(te, tv, xs, w1, b1r)


def _ffn2_body(te_ref, tv_ref, h_ref, w2_ref, b2_ref, ps_ref, ys_ref, w2bf):
    k = pl.program_id(0)

    @pl.when(_new_expert(te_ref, k))
    def _():
        w2bf[...] = w2_ref[0].astype(jnp.bfloat16)

    @pl.when(tv_ref[k] == 1)
    def _():
        y = jnp.dot(h_ref[...], w2bf[...], preferred_element_type=jnp.float32)
        y = y + b2_ref[0]
        ys_ref[...] = y * ps_ref[:, 0:1]


def _ffn2(te, tv, h, w2, b2r, ps):
    gs = pltpu.PrefetchScalarGridSpec(
        num_scalar_prefetch=2,
        grid=(NTILES,),
        in_specs=[
            pl.BlockSpec((TM, D_FF), lambda k, te, tv: (k, 0)),
            pl.BlockSpec((1, D_FF, D_MODEL), lambda k, te, tv: (te[k], 0, 0)),
            pl.BlockSpec((1, 1, D_MODEL), lambda k, te, tv: (te[k], 0, 0)),
            pl.BlockSpec((TM, 128), lambda k, te, tv: (k, 0)),
        ],
        out_specs=pl.BlockSpec((TM, D_MODEL), lambda k, te, tv: (k, 0)),
        scratch_shapes=[pltpu.VMEM((D_FF, D_MODEL), jnp.bfloat16)],
    )
    return pl.pallas_call(
        _ffn2_body,
        grid_spec=gs,
        out_shape=jax.ShapeDtypeStruct((NROWS, D_MODEL), jnp.float32),
    )(te, tv, h, w2, b2r, ps)


def kernel(x, Wg, bg, W1, b1, W2, b2):
    xf = x.reshape(NTOK, D_MODEL)
    pos, p, te, tv, aux = _plan(xf, Wg, bg.reshape(NUM_EXPERTS, 1))
    te1 = te.reshape(NTILES)
    tv1 = tv.reshape(NTILES)
    p16 = jnp.broadcast_to(p.reshape(NTOK, 1), (NTOK, 128))
    xs, ps = _dispatch(xf, p16, pos.reshape(NTOK))
    h = _ffn1(te1, tv1, xs, W1, b1.reshape(NUM_EXPERTS, 1, D_FF))
    ys = _ffn2(te1, tv1, h, W2, b2.reshape(NUM_EXPERTS, 1, D_MODEL), ps)
    out = _combine(ys, pos.reshape(NTOK))
    return out.reshape(x.shape), aux.reshape(())
